# baseline (speedup 1.0000x reference)
import jax
import jax.numpy as jnp
from jax.experimental import pallas as pl

DIM = 1024
K = 256
C = 8
CK = C * K
BT = 256


def _c2_kernel(to_ref, c2_ref):
    sq = to_ref[...] * to_ref[...]
    ones = jnp.ones((8, DIM), jnp.float32)
    c2 = jax.lax.dot_general(
        ones, sq, (((1,), (1,)), ((), ())),
        preferred_element_type=jnp.float32,
        precision=jax.lax.Precision.HIGHEST)
    c2_ref[...] = c2[0:1]


def _quant_kernel(x_ref, w_ref, b_ref, to_ref, c2_ref, out_ref):
    x = x_ref[...]
    to = to_ref[...]
    logits = jax.lax.dot_general(
        x, w_ref[...], (((1,), (1,)), ((), ())),
        preferred_element_type=jnp.float32) + b_ref[...]

    idx = []
    for c in range(C):
        sl = logits[:, c * K:(c + 1) * K]
        idx.append(jnp.argmax(sl, axis=1, keepdims=True))

    c2 = c2_ref[...]
    iota_k = jax.lax.broadcasted_iota(jnp.int32, (BT, K), 1)

    hi = to.astype(jnp.bfloat16)
    r1 = to - hi.astype(jnp.float32)
    mid = r1.astype(jnp.bfloat16)
    lo = (r1 - mid.astype(jnp.float32)).astype(jnp.bfloat16)

    def _gather(oh_bf, tab, c):
        return jax.lax.dot_general(
            oh_bf, tab[c * K:(c + 1) * K, :], (((1,), (0,)), ((), ())),
            preferred_element_type=jnp.float32)

    for _ in range(2):
        curs = []
        for c in range(C):
            oh = (iota_k == idx[c]).astype(jnp.bfloat16)
            curs.append((_gather(oh, hi, c) + _gather(oh, mid, c))
                        + _gather(oh, lo, c))
        x_err = (((curs[0] + curs[1]) + (curs[2] + curs[3]))
                 + ((curs[4] + curs[5]) + (curs[6] + curs[7]))) - x
        nidx = []
        for c in range(C):
            a_c = x_err - curs[c]
            a2_c = jnp.sum(a_c * a_c, axis=1, keepdims=True)
            ac = jax.lax.dot_general(
                a_c, to[c * K:(c + 1) * K, :], (((1,), (1,)), ((), ())),
                preferred_element_type=jnp.float32)
            score = (a2_c + c2[:, c * K:(c + 1) * K]) + 2.0 * ac
            nidx.append(jnp.argmin(score, axis=1, keepdims=True))
        idx = nidx

    iota_c = jax.lax.broadcasted_iota(jnp.int32, (BT, C), 1)
    out = jnp.zeros((BT, C), jnp.int32)
    for c in range(C):
        out = jnp.where(iota_c == c, idx[c], out)
    out_ref[...] = out


def kernel(x, W, b, to_output):
    B = x.shape[0]
    nblk = B // BT
    b2 = b.reshape(1, CK)
    c2 = pl.pallas_call(
        _c2_kernel,
        out_shape=jax.ShapeDtypeStruct((1, CK), jnp.float32),
    )(to_output)
    return pl.pallas_call(
        _quant_kernel,
        grid=(nblk,),
        in_specs=[
            pl.BlockSpec((BT, DIM), lambda i: (i, 0)),
            pl.BlockSpec((CK, DIM), lambda i: (0, 0)),
            pl.BlockSpec((1, CK), lambda i: (0, 0)),
            pl.BlockSpec((CK, DIM), lambda i: (0, 0)),
            pl.BlockSpec((1, CK), lambda i: (0, 0)),
        ],
        out_specs=pl.BlockSpec((BT, C), lambda i: (i, 0)),
        out_shape=jax.ShapeDtypeStruct((B, C), jnp.int32),
    )(x, W, b2, to_output, c2)


# speedup vs baseline: 1.1483x; 1.1483x over previous
import jax
import jax.numpy as jnp
from jax.experimental import pallas as pl

DIM = 1024
K = 256
C = 8
CK = C * K
BT = 512


def _c2_kernel(to_ref, c2_ref):
    sq = to_ref[...] * to_ref[...]
    ones = jnp.ones((8, DIM), jnp.float32)
    c2 = jax.lax.dot_general(
        ones, sq, (((1,), (1,)), ((), ())),
        preferred_element_type=jnp.float32,
        precision=jax.lax.Precision.HIGHEST)
    c2_ref[...] = c2[0:1]


def _quant_kernel(x_ref, w_ref, b_ref, to_ref, c2_ref, out_ref):
    x = x_ref[...]
    to = to_ref[...]
    logits = jax.lax.dot_general(
        x, w_ref[...], (((1,), (1,)), ((), ())),
        preferred_element_type=jnp.float32) + b_ref[...]

    iota_k = jax.lax.broadcasted_iota(jnp.int32, (BT, K), 1)
    idx = []
    for c in range(C):
        sl = logits[:, c * K:(c + 1) * K]
        mx = jnp.max(sl, axis=1, keepdims=True)
        idx.append(jnp.min(jnp.where(sl == mx, iota_k, K),
                           axis=1, keepdims=True))

    c2 = c2_ref[...]

    hi = to.astype(jnp.bfloat16)
    r1 = to - hi.astype(jnp.float32)
    mid = r1.astype(jnp.bfloat16)
    lo = (r1 - mid.astype(jnp.float32)).astype(jnp.bfloat16)

    def _gather(oh_bf, tab, c):
        return jax.lax.dot_general(
            oh_bf, tab[c * K:(c + 1) * K, :], (((1,), (0,)), ((), ())),
            preferred_element_type=jnp.float32)

    for _ in range(2):
        curs = []
        for c in range(C):
            oh = (iota_k == idx[c]).astype(jnp.bfloat16)
            curs.append((_gather(oh, hi, c) + _gather(oh, mid, c))
                        + _gather(oh, lo, c))
        x_err = (((curs[0] + curs[1]) + (curs[2] + curs[3]))
                 + ((curs[4] + curs[5]) + (curs[6] + curs[7]))) - x
        nidx = []
        for c in range(C):
            a_c = x_err - curs[c]
            a2_c = jnp.sum(a_c * a_c, axis=1, keepdims=True)
            ac = jax.lax.dot_general(
                a_c, to[c * K:(c + 1) * K, :], (((1,), (1,)), ((), ())),
                preferred_element_type=jnp.float32)
            score = (a2_c + c2[:, c * K:(c + 1) * K]) + 2.0 * ac
            mn = jnp.min(score, axis=1, keepdims=True)
            nidx.append(jnp.min(jnp.where(score == mn, iota_k, K),
                                axis=1, keepdims=True))
        idx = nidx

    iota_c = jax.lax.broadcasted_iota(jnp.int32, (BT, C), 1)
    out = jnp.zeros((BT, C), jnp.int32)
    for c in range(C):
        out = jnp.where(iota_c == c, idx[c], out)
    out_ref[...] = out


def kernel(x, W, b, to_output):
    B = x.shape[0]
    nblk = B // BT
    b2 = b.reshape(1, CK)
    c2 = pl.pallas_call(
        _c2_kernel,
        out_shape=jax.ShapeDtypeStruct((1, CK), jnp.float32),
    )(to_output)
    return pl.pallas_call(
        _quant_kernel,
        grid=(nblk,),
        in_specs=[
            pl.BlockSpec((BT, DIM), lambda i: (i, 0)),
            pl.BlockSpec((CK, DIM), lambda i: (0, 0)),
            pl.BlockSpec((1, CK), lambda i: (0, 0)),
            pl.BlockSpec((CK, DIM), lambda i: (0, 0)),
            pl.BlockSpec((1, CK), lambda i: (0, 0)),
        ],
        out_specs=pl.BlockSpec((BT, C), lambda i: (i, 0)),
        out_shape=jax.ShapeDtypeStruct((B, C), jnp.int32),
    )(x, W, b2, to_output, c2)
